# baseline (speedup 1.0000x reference)
import jax
import jax.numpy as jnp
from jax import lax
from jax.experimental import pallas as pl
from jax.experimental.pallas import tpu as pltpu
from jax.experimental.pallas import tpu_sc as plsc

NUM_ITEMS = 1000000
EMBED_DIM = 32
BATCH = 16384

NC = 2
NS = 16
NW = NC * NS
B_PER_W = BATCH // NW
CHUNK = 128
N_CHUNKS = B_PER_W // CHUNK


def _sc_body(table_hbm, idx_hbm, out_hbm, idx_v, rows_t, rows_c, out_v, sem):
    wid = lax.axis_index("s") * NC + lax.axis_index("c")

    pltpu.sync_copy(idx_hbm.at[wid], idx_v)

    copies = []
    for j in range(N_CHUNKS):
        d = pltpu.make_async_copy(
            table_hbm.at[idx_v.at[j]], rows_t.at[pl.ds(j * CHUNK, CHUNK)], sem)
        d.start()
        copies.append(d)
    for j in range(N_CHUNKS):
        d = pltpu.make_async_copy(
            table_hbm.at[idx_v.at[N_CHUNKS + j]],
            rows_c.at[pl.ds(j * CHUNK, CHUNK)], sem)
        d.start()
        copies.append(d)
    for d in copies:
        d.wait()

    lane = lax.iota(jnp.int32, 16)

    def block(blk, carry):
        base = blk * 16
        rid = base + lane
        acc = jnp.zeros((16,), jnp.float32)
        for dim in range(EMBED_DIM):
            cid = jnp.full((16,), dim, jnp.int32)
            tv = plsc.load_gather(rows_t, [rid, cid])
            cv = plsc.load_gather(rows_c, [rid, cid])
            acc = acc + tv * cv
        out_v[pl.ds(base, 16)] = 1.0 / (1.0 + jnp.exp(-acc))
        return carry

    lax.fori_loop(0, B_PER_W // 16, block, 0)

    pltpu.sync_copy(out_v, out_hbm.at[pl.ds(wid * B_PER_W, B_PER_W)])


_mesh = plsc.VectorSubcoreMesh(
    core_axis_name="c", subcore_axis_name="s", num_cores=NC, num_subcores=NS)

_sc_kernel = pl.kernel(
    _sc_body,
    out_type=jax.ShapeDtypeStruct((BATCH,), jnp.float32),
    mesh=_mesh,
    scratch_types=[
        pltpu.VMEM((2 * N_CHUNKS, CHUNK), jnp.int32),
        pltpu.VMEM((B_PER_W, EMBED_DIM), jnp.float32),
        pltpu.VMEM((B_PER_W, EMBED_DIM), jnp.float32),
        pltpu.VMEM((B_PER_W,), jnp.float32),
        pltpu.SemaphoreType.DMA,
    ],
    compiler_params=pltpu.CompilerParams(
        needs_layout_passes=False, use_tc_tiling_on_sc=False),
)


def kernel(target_items, context_items, embedding_table):
    t = target_items.astype(jnp.int32).reshape(NW, N_CHUNKS, CHUNK)
    c = context_items.astype(jnp.int32).reshape(NW, N_CHUNKS, CHUNK)
    idx = jnp.concatenate([t, c], axis=1)
    return _sc_kernel(embedding_table, idx)


# speedup vs baseline: 1.3726x; 1.3726x over previous
import jax
import jax.numpy as jnp
from jax import lax
from jax.experimental import pallas as pl
from jax.experimental.pallas import tpu as pltpu
from jax.experimental.pallas import tpu_sc as plsc

NUM_ITEMS = 1000000
EMBED_DIM = 32
BATCH = 16384

NC, NS = 2, 16
NW = NC * NS
B_PER_W = BATCH // NW
WAVE = 16
N_WAVES = B_PER_W // WAVE

_mesh = plsc.VectorSubcoreMesh(
    core_axis_name="c", subcore_axis_name="s", num_cores=NC, num_subcores=NS)


def _body(table_hbm, idx_hbm, out_hbm, idx_v, bt, bc, out_v, sem_t, sem_c):
    wid = lax.axis_index("s") * NC + lax.axis_index("c")
    pltpu.sync_copy(idx_hbm.at[wid], idx_v)

    lane = lax.iota(jnp.int32, 16)

    def wave(w, carry):
        base = w * WAVE
        vt = idx_v[0, pl.ds(base, 16)]
        vc = idx_v[1, pl.ds(base, 16)]
        bt_rows = 8 * lax.shift_right_logical(vt, 3)
        bc_rows = 8 * lax.shift_right_logical(vc, 3)
        for k in range(WAVE):
            rt = pl.multiple_of(bt_rows[k], 8)
            rc = pl.multiple_of(bc_rows[k], 8)
            pltpu.make_async_copy(
                table_hbm.at[pl.ds(rt, 8), :], bt.at[pl.ds(8 * k, 8), :], sem_t
            ).start()
            pltpu.make_async_copy(
                table_hbm.at[pl.ds(rc, 8), :], bc.at[pl.ds(8 * k, 8), :], sem_c
            ).start()
        pltpu.make_async_copy(table_hbm.at[pl.ds(0, 8 * WAVE), :], bt, sem_t).wait()
        pltpu.make_async_copy(table_hbm.at[pl.ds(0, 8 * WAVE), :], bc, sem_c).wait()

        st = vt & 7
        sc = vc & 7
        rowt = lane * 8 + st
        rowc = lane * 8 + sc
        acc = jnp.zeros((16,), jnp.float32)
        for d in range(EMBED_DIM):
            dvec = jnp.full((16,), d, jnp.int32)
            tv = plsc.load_gather(bt, [rowt, dvec])
            cv = plsc.load_gather(bc, [rowc, dvec])
            acc = acc + tv * cv
        out_v[pl.ds(base, 16)] = 1.0 / (1.0 + jnp.exp(-acc))
        return carry

    lax.fori_loop(0, N_WAVES, wave, 0)
    pltpu.sync_copy(out_v, out_hbm.at[pl.ds(wid * B_PER_W, B_PER_W)])


_k = pl.kernel(
    _body,
    out_type=jax.ShapeDtypeStruct((BATCH,), jnp.float32),
    mesh=_mesh,
    scratch_types=[
        pltpu.VMEM((2, B_PER_W), jnp.int32),
        pltpu.VMEM((8 * WAVE, EMBED_DIM), jnp.float32),
        pltpu.VMEM((8 * WAVE, EMBED_DIM), jnp.float32),
        pltpu.VMEM((B_PER_W,), jnp.float32),
        pltpu.SemaphoreType.DMA,
        pltpu.SemaphoreType.DMA,
    ],
    compiler_params=pltpu.CompilerParams(
        needs_layout_passes=False, use_tc_tiling_on_sc=True),
)


def kernel(target_items, context_items, embedding_table):
    t = target_items.astype(jnp.int32).reshape(NW, 1, B_PER_W)
    c = context_items.astype(jnp.int32).reshape(NW, 1, B_PER_W)
    idx = jnp.concatenate([t, c], axis=1)
    return _k(embedding_table, idx)


# speedup vs baseline: 1.9871x; 1.4477x over previous
import jax
import jax.numpy as jnp
from jax import lax
from jax.experimental import pallas as pl
from jax.experimental.pallas import tpu as pltpu
from jax.experimental.pallas import tpu_sc as plsc

NUM_ITEMS = 1000000
EMBED_DIM = 32
BATCH = 16384

NC, NS = 2, 16
NW = NC * NS
B_PER_W = BATCH // NW
WAVE = 8
N_WAVES = B_PER_W // WAVE

_mesh = plsc.VectorSubcoreMesh(
    core_axis_name="c", subcore_axis_name="s", num_cores=NC, num_subcores=NS)


def _body(tableT_hbm, idx_hbm, out_hbm, idx_v, bt, bc, out_v, sem_t, sem_c):
    wid = lax.axis_index("s") * NC + lax.axis_index("c")
    pltpu.sync_copy(idx_hbm.at[wid], idx_v)

    lane = lax.iota(jnp.int32, 16)

    def wave(w, acc):
        base = w * WAVE
        vt = idx_v[0, pl.ds(base, 16)]
        vc = idx_v[1, pl.ds(base, 16)]
        pt = 128 * lax.shift_right_logical(vt, 7)
        pc = 128 * lax.shift_right_logical(vc, 7)
        lt = vt & 127
        lc = vc & 127

        descs = []
        for k in range(WAVE):
            ot = pl.multiple_of(pt[k], 128)
            oc = pl.multiple_of(pc[k], 128)
            d1 = pltpu.make_async_copy(
                tableT_hbm.at[:, pl.ds(ot, 128)],
                bt.at[pl.ds(32 * k, 32), :], sem_t)
            d2 = pltpu.make_async_copy(
                tableT_hbm.at[:, pl.ds(oc, 128)],
                bc.at[pl.ds(32 * k, 32), :], sem_c)
            d1.start()
            d2.start()
            descs += [d1, d2]
        for d in descs:
            d.wait()

        for k in range(WAVE):
            lt_s = jnp.full((16,), lt[k], jnp.int32)
            lc_s = jnp.full((16,), lc[k], jnp.int32)
            tlo = plsc.load_gather(bt, [32 * k + lane, lt_s])
            thi = plsc.load_gather(bt, [32 * k + 16 + lane, lt_s])
            clo = plsc.load_gather(bc, [32 * k + lane, lc_s])
            chi = plsc.load_gather(bc, [32 * k + 16 + lane, lc_s])
            dot = jnp.sum(tlo * clo + thi * chi)
            slot = (w & 1) * 8 + k
            acc = jnp.where(lane == slot, dot, acc)

        @pl.when((w & 1) == 1)
        def _():
            out_v[pl.ds((w >> 1) * 16, 16)] = 1.0 / (1.0 + jnp.exp(-acc))

        return acc

    lax.fori_loop(0, N_WAVES, wave, jnp.zeros((16,), jnp.float32))
    pltpu.sync_copy(out_v, out_hbm.at[pl.ds(wid * B_PER_W, B_PER_W)])


_k = pl.kernel(
    _body,
    out_type=jax.ShapeDtypeStruct((BATCH,), jnp.float32),
    mesh=_mesh,
    scratch_types=[
        pltpu.VMEM((2, B_PER_W), jnp.int32),
        pltpu.VMEM((32 * WAVE, 128), jnp.float32),
        pltpu.VMEM((32 * WAVE, 128), jnp.float32),
        pltpu.VMEM((B_PER_W,), jnp.float32),
        pltpu.SemaphoreType.DMA,
        pltpu.SemaphoreType.DMA,
    ],
    compiler_params=pltpu.CompilerParams(
        needs_layout_passes=False, use_tc_tiling_on_sc=True),
)


def kernel(target_items, context_items, embedding_table):
    t = target_items.astype(jnp.int32).reshape(NW, 1, B_PER_W)
    c = context_items.astype(jnp.int32).reshape(NW, 1, B_PER_W)
    idx = jnp.concatenate([t, c], axis=1)
    tT = embedding_table.T
    return _k(tT, idx)


# speedup vs baseline: 2.0753x; 1.0444x over previous
import jax
import jax.numpy as jnp
from jax import lax
from jax.experimental import pallas as pl
from jax.experimental.pallas import tpu as pltpu
from jax.experimental.pallas import tpu_sc as plsc

NUM_ITEMS = 1000000
EMBED_DIM = 32
BATCH = 16384

NC, NS = 2, 16
NW = NC * NS
B_PER_W = BATCH // NW
WAVE = 4
N_WAVES = B_PER_W // WAVE
MAX_PANEL = 7812

_mesh = plsc.VectorSubcoreMesh(
    core_axis_name="c", subcore_axis_name="s", num_cores=NC, num_subcores=NS)


def _fire(tableT_hbm, idx_v, bt, bc, sem, w):
    vt = idx_v[0, pl.ds(w * WAVE, 16)]
    vc = idx_v[1, pl.ds(w * WAVE, 16)]
    pt = 128 * jnp.minimum(lax.shift_right_logical(vt, 7), MAX_PANEL)
    pc = 128 * jnp.minimum(lax.shift_right_logical(vc, 7), MAX_PANEL)
    for k in range(WAVE):
        ot = pl.multiple_of(pt[k], 128)
        oc = pl.multiple_of(pc[k], 128)
        pltpu.make_async_copy(
            tableT_hbm.at[:, pl.ds(ot, 128)],
            bt.at[:, pl.ds(128 * k, 128)], sem).start()
        pltpu.make_async_copy(
            tableT_hbm.at[:, pl.ds(oc, 128)],
            bc.at[:, pl.ds(128 * k, 128)], sem).start()


def _body(tableT_hbm, idx_hbm, out_hbm, idx_v,
          bt0, bc0, bt1, bc1, out_v, sem0, sem1):
    wid = lax.axis_index("s") * NC + lax.axis_index("c")
    pltpu.sync_copy(idx_hbm.at[wid], idx_v.at[:, pl.ds(0, B_PER_W)])

    lane = lax.iota(jnp.int32, 16)
    bufs = ((bt0, bc0, sem0), (bt1, bc1, sem1))

    _fire(tableT_hbm, idx_v, bt0, bc0, sem0, 0)

    def wave(w, acc):
        for p in range(2):

            @pl.when((w & 1) == p)
            def _():
                btp, bcp, semp = bufs[p]
                btn, bcn, semn = bufs[1 - p]

                @pl.when(w < N_WAVES - 1)
                def _():
                    _fire(tableT_hbm, idx_v, btn, bcn, semn, w + 1)

                pltpu.make_async_copy(
                    tableT_hbm.at[:, pl.ds(0, 128 * WAVE)], btp, semp).wait()
                pltpu.make_async_copy(
                    tableT_hbm.at[:, pl.ds(0, 128 * WAVE)], bcp, semp).wait()

        vt = idx_v[0, pl.ds(w * WAVE, 16)]
        vc = idx_v[1, pl.ds(w * WAVE, 16)]
        lt = vt & 127
        lc = vc & 127
        for p in range(2):
            btp, bcp, _ = bufs[p]
            for k in range(WAVE):
                lts = jnp.full((16,), lt[k], jnp.int32) + 128 * k
                lcs = jnp.full((16,), lc[k], jnp.int32) + 128 * k
                tlo = plsc.load_gather(btp, [lane, lts])
                thi = plsc.load_gather(btp, [16 + lane, lts])
                clo = plsc.load_gather(bcp, [lane, lcs])
                chi = plsc.load_gather(bcp, [16 + lane, lcs])
                dot = jnp.sum(tlo * clo + thi * chi)
                slot = (w & 3) * WAVE + k
                sel = ((w & 1) == p) & (lane == slot)
                acc = jnp.where(sel, dot, acc)

        @pl.when((w & 3) == 3)
        def _():
            out_v[pl.ds((w >> 2) * 16, 16)] = 1.0 / (1.0 + jnp.exp(-acc))

        return acc

    lax.fori_loop(0, N_WAVES, wave, jnp.zeros((16,), jnp.float32))
    pltpu.sync_copy(out_v, out_hbm.at[pl.ds(wid * B_PER_W, B_PER_W)])


_k = pl.kernel(
    _body,
    out_type=jax.ShapeDtypeStruct((BATCH,), jnp.float32),
    mesh=_mesh,
    scratch_types=[
        pltpu.VMEM((2, B_PER_W + 32), jnp.int32),
        pltpu.VMEM((EMBED_DIM, 128 * WAVE), jnp.float32),
        pltpu.VMEM((EMBED_DIM, 128 * WAVE), jnp.float32),
        pltpu.VMEM((EMBED_DIM, 128 * WAVE), jnp.float32),
        pltpu.VMEM((EMBED_DIM, 128 * WAVE), jnp.float32),
        pltpu.VMEM((B_PER_W,), jnp.float32),
        pltpu.SemaphoreType.DMA,
        pltpu.SemaphoreType.DMA,
    ],
    compiler_params=pltpu.CompilerParams(
        needs_layout_passes=False, use_tc_tiling_on_sc=True),
)


def kernel(target_items, context_items, embedding_table):
    t = target_items.astype(jnp.int32).reshape(NW, 1, B_PER_W)
    c = context_items.astype(jnp.int32).reshape(NW, 1, B_PER_W)
    idx = jnp.concatenate([t, c], axis=1)
    tT = embedding_table.T
    return _k(tT, idx)


# speedup vs baseline: 2.3337x; 1.1245x over previous
import jax
import jax.numpy as jnp
from jax import lax
from jax.experimental import pallas as pl
from jax.experimental.pallas import tpu as pltpu
from jax.experimental.pallas import tpu_sc as plsc

NUM_ITEMS = 1000000
EMBED_DIM = 32
BATCH = 16384

NC, NS = 2, 16
NW = NC * NS
B_PER_W = BATCH // NW
WAVE = 4
N_WAVES = B_PER_W // WAVE
MAX_PANEL = 7812

_mesh = plsc.VectorSubcoreMesh(
    core_axis_name="c", subcore_axis_name="s", num_cores=NC, num_subcores=NS)


def _fire(tableT_hbm, idx_v, bt, bc, sem, w):
    vt = idx_v[0, pl.ds(w * WAVE, 16)]
    vc = idx_v[1, pl.ds(w * WAVE, 16)]
    pt = 128 * jnp.minimum(lax.shift_right_logical(vt, 7), MAX_PANEL)
    pc = 128 * jnp.minimum(lax.shift_right_logical(vc, 7), MAX_PANEL)
    for k in range(WAVE):
        ot = pl.multiple_of(pt[k], 128)
        oc = pl.multiple_of(pc[k], 128)
        pltpu.make_async_copy(
            tableT_hbm.at[:, pl.ds(ot, 128)],
            bt.at[:, pl.ds(128 * k, 128)], sem).start()
        pltpu.make_async_copy(
            tableT_hbm.at[:, pl.ds(oc, 128)],
            bc.at[:, pl.ds(128 * k, 128)], sem).start()


def _body(tableT_hbm, idx_hbm, out_hbm, idx_v,
          bt0, bc0, bt1, bc1, bt2, bc2, out_v, sem0, sem1, sem2):
    wid = lax.axis_index("s") * NC + lax.axis_index("c")
    pltpu.sync_copy(idx_hbm.at[wid], idx_v.at[:, pl.ds(0, B_PER_W)])

    lane = lax.iota(jnp.int32, 16)
    bufs = ((bt0, bc0, sem0), (bt1, bc1, sem1), (bt2, bc2, sem2))

    _fire(tableT_hbm, idx_v, bt0, bc0, sem0, 0)
    _fire(tableT_hbm, idx_v, bt1, bc1, sem1, 1)

    def wave(w, carry):
        wm3 = lax.rem(w, 3)
        for p in range(3):

            @pl.when(wm3 == p)
            def _():
                btp, bcp, semp = bufs[p]
                btn, bcn, semn = bufs[(p + 2) % 3]

                @pl.when(w < N_WAVES - 2)
                def _():
                    _fire(tableT_hbm, idx_v, btn, bcn, semn, w + 2)

                pltpu.make_async_copy(
                    tableT_hbm.at[:, pl.ds(0, 128 * WAVE)], btp, semp).wait()
                pltpu.make_async_copy(
                    tableT_hbm.at[:, pl.ds(0, 128 * WAVE)], bcp, semp).wait()

                vt = idx_v[0, pl.ds(w * WAVE, 16)]
                vc = idx_v[1, pl.ds(w * WAVE, 16)]
                lt = vt & 127
                lc = vc & 127
                acc = out_v[pl.ds((w >> 2) * 16, 16)]
                for k in range(WAVE):
                    lts = jnp.full((16,), lt[k], jnp.int32) + 128 * k
                    lcs = jnp.full((16,), lc[k], jnp.int32) + 128 * k
                    tlo = plsc.load_gather(btp, [lane, lts])
                    thi = plsc.load_gather(btp, [16 + lane, lts])
                    clo = plsc.load_gather(bcp, [lane, lcs])
                    chi = plsc.load_gather(bcp, [16 + lane, lcs])
                    dot = jnp.sum(tlo * clo + thi * chi)
                    slot = (w & 3) * WAVE + k
                    acc = jnp.where(lane == slot, dot, acc)
                out_v[pl.ds((w >> 2) * 16, 16)] = acc

                @pl.when((w & 3) == 3)
                def _():
                    out_v[pl.ds((w >> 2) * 16, 16)] = (
                        1.0 / (1.0 + jnp.exp(-acc)))

        return carry

    lax.fori_loop(0, N_WAVES, wave, 0)
    pltpu.sync_copy(out_v, out_hbm.at[pl.ds(wid * B_PER_W, B_PER_W)])


_k = pl.kernel(
    _body,
    out_type=jax.ShapeDtypeStruct((BATCH,), jnp.float32),
    mesh=_mesh,
    scratch_types=[
        pltpu.VMEM((2, B_PER_W + 32), jnp.int32),
        pltpu.VMEM((EMBED_DIM, 128 * WAVE), jnp.float32),
        pltpu.VMEM((EMBED_DIM, 128 * WAVE), jnp.float32),
        pltpu.VMEM((EMBED_DIM, 128 * WAVE), jnp.float32),
        pltpu.VMEM((EMBED_DIM, 128 * WAVE), jnp.float32),
        pltpu.VMEM((EMBED_DIM, 128 * WAVE), jnp.float32),
        pltpu.VMEM((EMBED_DIM, 128 * WAVE), jnp.float32),
        pltpu.VMEM((B_PER_W,), jnp.float32),
        pltpu.SemaphoreType.DMA,
        pltpu.SemaphoreType.DMA,
        pltpu.SemaphoreType.DMA,
    ],
    compiler_params=pltpu.CompilerParams(
        needs_layout_passes=False, use_tc_tiling_on_sc=True),
)


def kernel(target_items, context_items, embedding_table):
    t = target_items.astype(jnp.int32).reshape(NW, 1, B_PER_W)
    c = context_items.astype(jnp.int32).reshape(NW, 1, B_PER_W)
    idx = jnp.concatenate([t, c], axis=1)
    tT = embedding_table.T
    return _k(tT, idx)


# speedup vs baseline: 2.3396x; 1.0025x over previous
import jax
import jax.numpy as jnp
from jax import lax
from jax.experimental import pallas as pl
from jax.experimental.pallas import tpu as pltpu
from jax.experimental.pallas import tpu_sc as plsc

NUM_ITEMS = 1000000
EMBED_DIM = 32
BATCH = 16384

NC, NS = 2, 16
NW = NC * NS
B_PER_W = BATCH // NW
WAVE = 2
N_WAVES = B_PER_W // WAVE
NSETS = 4
MAX_PANEL = 7812

_mesh = plsc.VectorSubcoreMesh(
    core_axis_name="c", subcore_axis_name="s", num_cores=NC, num_subcores=NS)


def _fire(tableT_hbm, idx_v, bt, bc, sem, w):
    vt = idx_v[0, pl.ds(w * WAVE, 16)]
    vc = idx_v[1, pl.ds(w * WAVE, 16)]
    pt = 128 * jnp.minimum(lax.shift_right_logical(vt, 7), MAX_PANEL)
    pc = 128 * jnp.minimum(lax.shift_right_logical(vc, 7), MAX_PANEL)
    for k in range(WAVE):
        ot = pl.multiple_of(pt[k], 128)
        oc = pl.multiple_of(pc[k], 128)
        pltpu.make_async_copy(
            tableT_hbm.at[:, pl.ds(ot, 128)],
            bt.at[:, pl.ds(128 * k, 128)], sem).start()
        pltpu.make_async_copy(
            tableT_hbm.at[:, pl.ds(oc, 128)],
            bc.at[:, pl.ds(128 * k, 128)], sem).start()


def _body(tableT_hbm, idx_hbm, out_hbm, idx_v,
          bt0, bc0, bt1, bc1, bt2, bc2, bt3, bc3, out_v,
          sem0, sem1, sem2, sem3):
    wid = lax.axis_index("s") * NC + lax.axis_index("c")
    pltpu.sync_copy(idx_hbm.at[wid], idx_v.at[:, pl.ds(0, B_PER_W)])

    lane = lax.iota(jnp.int32, 16)
    bufs = ((bt0, bc0, sem0), (bt1, bc1, sem1),
            (bt2, bc2, sem2), (bt3, bc3, sem3))

    for i in range(NSETS - 1):
        _fire(tableT_hbm, idx_v, bufs[i][0], bufs[i][1], bufs[i][2], i)

    def wave(w, carry):
        wp = w & (NSETS - 1)
        for p in range(NSETS):

            @pl.when(wp == p)
            def _():
                btp, bcp, semp = bufs[p]
                btn, bcn, semn = bufs[(p + NSETS - 1) % NSETS]

                @pl.when(w < N_WAVES - (NSETS - 1))
                def _():
                    _fire(tableT_hbm, idx_v, btn, bcn, semn, w + NSETS - 1)

                pltpu.make_async_copy(
                    tableT_hbm.at[:, pl.ds(0, 128 * WAVE)], btp, semp).wait()
                pltpu.make_async_copy(
                    tableT_hbm.at[:, pl.ds(0, 128 * WAVE)], bcp, semp).wait()

                vt = idx_v[0, pl.ds(w * WAVE, 16)]
                vc = idx_v[1, pl.ds(w * WAVE, 16)]
                lt = vt & 127
                lc = vc & 127
                acc = out_v[pl.ds((w >> 3) * 16, 16)]
                for k in range(WAVE):
                    lts = jnp.full((16,), lt[k], jnp.int32) + 128 * k
                    lcs = jnp.full((16,), lc[k], jnp.int32) + 128 * k
                    tlo = plsc.load_gather(btp, [lane, lts])
                    thi = plsc.load_gather(btp, [16 + lane, lts])
                    clo = plsc.load_gather(bcp, [lane, lcs])
                    chi = plsc.load_gather(bcp, [16 + lane, lcs])
                    dot = jnp.sum(tlo * clo + thi * chi)
                    slot = (w & 7) * WAVE + k
                    acc = jnp.where(lane == slot, dot, acc)
                out_v[pl.ds((w >> 3) * 16, 16)] = acc

                @pl.when((w & 7) == 7)
                def _():
                    out_v[pl.ds((w >> 3) * 16, 16)] = (
                        1.0 / (1.0 + jnp.exp(-acc)))

        return carry

    lax.fori_loop(0, N_WAVES, wave, 0)
    pltpu.sync_copy(out_v, out_hbm.at[pl.ds(wid * B_PER_W, B_PER_W)])


_panel = pltpu.VMEM((EMBED_DIM, 128 * WAVE), jnp.float32)

_k = pl.kernel(
    _body,
    out_type=jax.ShapeDtypeStruct((BATCH,), jnp.float32),
    mesh=_mesh,
    scratch_types=[
        pltpu.VMEM((2, B_PER_W + 32), jnp.int32),
        _panel, _panel,
        _panel, _panel,
        _panel, _panel,
        _panel, _panel,
        pltpu.VMEM((B_PER_W,), jnp.float32),
        pltpu.SemaphoreType.DMA,
        pltpu.SemaphoreType.DMA,
        pltpu.SemaphoreType.DMA,
        pltpu.SemaphoreType.DMA,
    ],
    compiler_params=pltpu.CompilerParams(
        needs_layout_passes=False, use_tc_tiling_on_sc=True),
)


def kernel(target_items, context_items, embedding_table):
    t = target_items.astype(jnp.int32).reshape(NW, 1, B_PER_W)
    c = context_items.astype(jnp.int32).reshape(NW, 1, B_PER_W)
    idx = jnp.concatenate([t, c], axis=1)
    tT = embedding_table.T
    return _k(tT, idx)
